# baseline (speedup 1.0000x reference)
import functools

import jax
import jax.numpy as jnp
from jax.experimental import pallas as pl
from jax.experimental.pallas import tpu as pltpu

EPS = 1e-5
_DT = jnp.bfloat16


def _round_up(v, m):
    return (v + m - 1) // m * m


def _pack_w_s2(w):
    cout, cin, _, _ = w.shape
    wt = jnp.transpose(w, (2, 3, 1, 0))
    wt = wt.reshape(2, 2, 2, 2, cin, cout)
    wt = wt.transpose(0, 2, 1, 3, 4, 5)
    return wt.reshape(4, 4 * cin, cout).astype(_DT)


def _pack_w1(w):
    return _pack_w_s2(w).reshape(-1, w.shape[0])


def _pack_w_s1(w):
    cout, cin, _, _ = w.shape
    coutp = _round_up(cout, 128)
    wt = jnp.transpose(w, (2, 3, 1, 0))
    wt = jnp.pad(wt, ((0, 0), (0, 0), (0, 0), (0, coutp - cout)))
    return wt.reshape(16, cin, coutp).astype(_DT)


def _fold_value(z):
    bo, h, w, c = z.shape
    hh = h // 2
    zp = jnp.pad(z, ((0, 0), (1, 1), (1, 1), (0, 0)))
    v = zp.reshape(bo, hh + 1, 2, hh + 1, 2, c)
    return jnp.concatenate(
        [v[:, :, rp, :, cp, :] for rp in range(2) for cp in range(2)],
        axis=-1)


def _s2_taps(x, wt, bo, hh):
    c4 = x.shape[-1]
    xa = x[:, :, 0:hh, :]
    xb = x[:, :, 1:hh + 1, :]
    m = bo * hh * hh
    acc = jnp.zeros((m, wt.shape[2]), jnp.float32)
    for a in range(2):
        ta = xa[:, a:a + hh].reshape(m, c4)
        tb = xb[:, a:a + hh].reshape(m, c4)
        acc = acc + jnp.dot(ta, wt[a * 2], preferred_element_type=jnp.float32)
        acc = acc + jnp.dot(tb, wt[a * 2 + 1], preferred_element_type=jnp.float32)
    return acc


def _conv12_kernel(x_ref, w1_ref, w2_ref, y_ref, s_ref, q_ref,
                   *, bo, hh, c1, c2):
    ho = 2 * hh
    m1 = bo * ho * ho
    x1 = _fold_value(x_ref[...])
    k0 = x1.shape[-1]
    xa = x1[:, :, 0:ho, :]
    xb = x1[:, :, 1:ho + 1, :]
    patch = jnp.concatenate(
        [(xa if b == 0 else xb)[:, a:a + ho, :, :].reshape(m1, k0)
         for a in range(2) for b in range(2)], axis=-1)
    y1 = jnp.dot(patch, w1_ref[...], preferred_element_type=jnp.float32)
    y1 = jnp.maximum(y1, 0.0).astype(_DT)
    y1 = y1.reshape(bo, ho, ho, c1)

    x2 = _fold_value(y1)
    acc = _s2_taps(x2, w2_ref[...], bo, hh)
    y_ref[...] = acc.reshape(bo, hh, hh, c2).astype(y_ref.dtype)
    s_ref[0, 0] = jnp.sum(acc, axis=0)
    q_ref[0, 0] = jnp.sum(acc * acc, axis=0)


def _conv12(xt, w1m, w2t, bo):
    n, h, _, cin = xt.shape
    hh = h // 4
    ho = h // 2
    k1, c1 = w1m.shape
    c2 = w2t.shape[2]
    bo = min(bo, n)
    nb = n // bo
    kern = functools.partial(_conv12_kernel, bo=bo, hh=hh, c1=c1, c2=c2)
    flops = 2 * n * ho * ho * k1 * c1 + 2 * n * hh * hh * 4 * c1 * c2
    bytes_acc = 2 * (xt.size + w1m.size + w2t.size + n * hh * hh * c2)
    return pl.pallas_call(
        kern,
        grid=(nb,),
        in_specs=[
            pl.BlockSpec((bo, h, h, cin), lambda i: (i, 0, 0, 0)),
            pl.BlockSpec((k1, c1), lambda i: (0, 0)),
            pl.BlockSpec((4, 4 * c1, c2), lambda i: (0, 0, 0)),
        ],
        out_specs=(
            pl.BlockSpec((bo, hh, hh, c2), lambda i: (i, 0, 0, 0)),
            pl.BlockSpec((1, 1, c2), lambda i: (i, 0, 0)),
            pl.BlockSpec((1, 1, c2), lambda i: (i, 0, 0)),
        ),
        out_shape=(
            jax.ShapeDtypeStruct((n, hh, hh, c2), _DT),
            jax.ShapeDtypeStruct((nb, 1, c2), jnp.float32),
            jax.ShapeDtypeStruct((nb, 1, c2), jnp.float32),
        ),
        compiler_params=pltpu.CompilerParams(
            dimension_semantics=("parallel",),
            vmem_limit_bytes=100 * 1024 * 1024,
        ),
        cost_estimate=pl.CostEstimate(flops=flops, transcendentals=0,
                                      bytes_accessed=bytes_acc),
    )(xt, w1m, w2t)


def _bnconv_kernel(y_ref, sc_ref, sh_ref, w_ref, o_ref, s_ref, q_ref, *, bo):
    z = y_ref[...]
    z = jnp.maximum(z * sc_ref[0] + sh_ref[0], 0.0).astype(_DT)
    hh = z.shape[1] // 2
    x = _fold_value(z)
    acc = _s2_taps(x, w_ref[...], bo, hh)
    co = acc.shape[-1]
    o_ref[...] = acc.reshape(bo, hh, hh, co).astype(o_ref.dtype)
    s_ref[0, 0] = jnp.sum(acc, axis=0)
    q_ref[0, 0] = jnp.sum(acc * acc, axis=0)


def _bnconv(y, scale, shift, wt, bo):
    n, h, _, c = y.shape
    hh = h // 2
    co = wt.shape[2]
    bo = min(bo, n)
    nb = n // bo
    kern = functools.partial(_bnconv_kernel, bo=bo)
    flops = 2 * n * hh * hh * 4 * c * co
    bytes_acc = 2 * (y.size + wt.size + n * hh * hh * co)
    return pl.pallas_call(
        kern,
        grid=(nb,),
        in_specs=[
            pl.BlockSpec((bo, h, h, c), lambda i: (i, 0, 0, 0)),
            pl.BlockSpec((1, c), lambda i: (0, 0)),
            pl.BlockSpec((1, c), lambda i: (0, 0)),
            pl.BlockSpec((4, 4 * c, co), lambda i: (0, 0, 0)),
        ],
        out_specs=(
            pl.BlockSpec((bo, hh, hh, co), lambda i: (i, 0, 0, 0)),
            pl.BlockSpec((1, 1, co), lambda i: (i, 0, 0)),
            pl.BlockSpec((1, 1, co), lambda i: (i, 0, 0)),
        ),
        out_shape=(
            jax.ShapeDtypeStruct((n, hh, hh, co), _DT),
            jax.ShapeDtypeStruct((nb, 1, co), jnp.float32),
            jax.ShapeDtypeStruct((nb, 1, co), jnp.float32),
        ),
        compiler_params=pltpu.CompilerParams(
            dimension_semantics=("parallel",),
            vmem_limit_bytes=100 * 1024 * 1024,
        ),
        cost_estimate=pl.CostEstimate(flops=flops, transcendentals=0,
                                      bytes_accessed=bytes_acc),
    )(y, scale, shift, wt)


def _conv5_kernel(y_ref, sc_ref, sh_ref, w_ref, o_ref, *, bo):
    z = y_ref[...]
    z = jnp.maximum(z * sc_ref[0] + sh_ref[0], 0.0).astype(_DT)
    h = z.shape[1]
    c = z.shape[-1]
    ho = h - 1
    zp = jnp.pad(z, ((0, 0), (1, 1), (1, 1), (0, 0)))
    m = bo * ho * ho
    co = w_ref.shape[2]
    acc = jnp.zeros((m, co), jnp.float32)
    for t in range(16):
        di, dj = divmod(t, 4)
        tap = zp[:, di:di + ho, dj:dj + ho, :].reshape(m, c)
        acc = acc + jnp.dot(tap, w_ref[t], preferred_element_type=jnp.float32)
    o_ref[...] = acc.reshape(bo, ho, ho, co)


def _conv5(y, scale, shift, wt, bo):
    n, h, _, c = y.shape
    ho = h - 1
    co = wt.shape[2]
    bo = min(bo, n)
    nb = n // bo
    kern = functools.partial(_conv5_kernel, bo=bo)
    flops = 2 * n * ho * ho * 16 * c * co
    bytes_acc = 2 * (y.size + wt.size) + 4 * n * ho * ho * co
    return pl.pallas_call(
        kern,
        grid=(nb,),
        in_specs=[
            pl.BlockSpec((bo, h, h, c), lambda i: (i, 0, 0, 0)),
            pl.BlockSpec((1, c), lambda i: (0, 0)),
            pl.BlockSpec((1, c), lambda i: (0, 0)),
            pl.BlockSpec((16, c, co), lambda i: (0, 0, 0)),
        ],
        out_specs=pl.BlockSpec((bo, ho, ho, co), lambda i: (i, 0, 0, 0)),
        out_shape=jax.ShapeDtypeStruct((n, ho, ho, co), jnp.float32),
        compiler_params=pltpu.CompilerParams(
            dimension_semantics=("parallel",),
            vmem_limit_bytes=100 * 1024 * 1024,
        ),
        cost_estimate=pl.CostEstimate(flops=flops, transcendentals=0,
                                      bytes_accessed=bytes_acc),
    )(y, scale, shift, wt)


def _bn_affine(s, q, count, gamma, beta):
    mean = jnp.sum(s, axis=(0, 1)) / count
    var = jnp.maximum(jnp.sum(q, axis=(0, 1)) / count - mean * mean, 0.0)
    scale = gamma * jax.lax.rsqrt(var + EPS)
    shift = beta - mean * scale
    return scale.reshape(1, -1), shift.reshape(1, -1)


def kernel(x, w1, w2, w3, w4, w5, g2, b2, g3, b3, g4, b4):
    n = x.shape[0]

    xt = jnp.transpose(x, (0, 2, 3, 1)).astype(_DT)
    y2, s2, q2 = _conv12(xt, _pack_w1(w1), _pack_w_s2(w2), 8)
    a2 = _bn_affine(s2, q2, n * y2.shape[1] * y2.shape[2], g2, b2)

    y3, s3, q3 = _bnconv(y2, *a2, _pack_w_s2(w3), 32)
    a3 = _bn_affine(s3, q3, n * y3.shape[1] * y3.shape[2], g3, b3)

    y4, s4, q4 = _bnconv(y3, *a3, _pack_w_s2(w4), 64)
    a4 = _bn_affine(s4, q4, n * y4.shape[1] * y4.shape[2], g4, b4)

    y5 = _conv5(y4, *a4, _pack_w_s1(w5), 64)

    out = y5[..., :1]
    return jnp.transpose(out, (0, 3, 1, 2))


# speedup vs baseline: 3.6890x; 3.6890x over previous
import functools

import jax
import jax.numpy as jnp
from jax.experimental import pallas as pl
from jax.experimental.pallas import tpu as pltpu

EPS = 1e-5
_DT = jnp.bfloat16


def _round_up(v, m):
    return (v + m - 1) // m * m


def _pack_w_s2(w):
    cout, cin, _, _ = w.shape
    wt = jnp.transpose(w, (2, 3, 1, 0))
    wt = wt.reshape(2, 2, 2, 2, cin, cout)
    wt = wt.transpose(0, 2, 1, 3, 4, 5)
    return wt.reshape(4, 4 * cin, cout).astype(_DT)


def _pack_w1(w):
    return _pack_w_s2(w).reshape(-1, w.shape[0])


def _pack_w_s1(w):
    cout, cin, _, _ = w.shape
    coutp = _round_up(cout, 128)
    wt = jnp.transpose(w, (2, 3, 1, 0))
    wt = jnp.pad(wt, ((0, 0), (0, 0), (0, 0), (0, coutp - cout)))
    return wt.reshape(16, cin, coutp).astype(_DT)


def _fold_value(z):
    bo, h, w, c = z.shape
    hh = h // 2
    zp = jnp.pad(z, ((0, 0), (1, 1), (1, 1), (0, 0)))
    v = zp.reshape(bo, hh + 1, 2, hh + 1, 2, c)
    return jnp.concatenate(
        [v[:, :, rp, :, cp, :] for rp in range(2) for cp in range(2)],
        axis=-1)


def _s2_taps(x, wt, bo, hh):
    c4 = x.shape[-1]
    xa = x[:, :, 0:hh, :]
    xb = x[:, :, 1:hh + 1, :]
    m = bo * hh * hh
    acc = jnp.zeros((m, wt.shape[2]), jnp.float32)
    for a in range(2):
        ta = xa[:, a:a + hh].reshape(m, c4)
        tb = xb[:, a:a + hh].reshape(m, c4)
        acc = acc + jnp.dot(ta, wt[a * 2], preferred_element_type=jnp.float32)
        acc = acc + jnp.dot(tb, wt[a * 2 + 1], preferred_element_type=jnp.float32)
    return acc


def _host_x1(x):
    xt = jnp.transpose(x, (0, 2, 3, 1)).astype(_DT)
    n, h, _, c = xt.shape
    hp = h // 2 + 1
    xp = jnp.pad(xt, ((0, 0), (1, 1), (1, 1), (0, 0)))
    x1 = xp.reshape(n, hp, 2, hp, 2, c).transpose(0, 1, 3, 2, 4, 5)
    x1 = x1.reshape(n, hp, hp, 4 * c)
    return x1[:, :, 0:-1, :], x1[:, :, 1:, :]


def _conv12_kernel(x1a_ref, x1b_ref, w1_ref, w2_ref, y_ref, s_ref, q_ref,
                   *, bo, hh, c1, c2):
    ho = 2 * hh
    m1 = bo * ho * ho
    k0 = x1a_ref.shape[-1]
    xa = x1a_ref[...]
    xb = x1b_ref[...]
    patch = jnp.concatenate(
        [(xa if b == 0 else xb)[:, a:a + ho, :, :].reshape(m1, k0)
         for a in range(2) for b in range(2)], axis=-1)
    y1 = jnp.dot(patch, w1_ref[...], preferred_element_type=jnp.float32)
    y1 = jnp.maximum(y1, 0.0).astype(_DT)
    y1 = y1.reshape(bo, ho, ho, c1)

    x2 = _fold_value(y1)
    acc = _s2_taps(x2, w2_ref[...], bo, hh)
    y_ref[...] = acc.reshape(bo, hh, hh, c2).astype(y_ref.dtype)
    s_ref[0, 0] = jnp.sum(acc, axis=0)
    q_ref[0, 0] = jnp.sum(acc * acc, axis=0)


def _conv12(x1a, x1b, w1m, w2t, bo):
    n, hp, ho, k0 = x1a.shape
    hh = ho // 2
    k1, c1 = w1m.shape
    c2 = w2t.shape[2]
    bo = min(bo, n)
    nb = n // bo
    kern = functools.partial(_conv12_kernel, bo=bo, hh=hh, c1=c1, c2=c2)
    flops = 2 * n * ho * ho * k1 * c1 + 2 * n * hh * hh * 4 * c1 * c2
    bytes_acc = 2 * (x1a.size + x1b.size + w1m.size + w2t.size
                     + n * hh * hh * c2)
    return pl.pallas_call(
        kern,
        grid=(nb,),
        in_specs=[
            pl.BlockSpec((bo, hp, ho, k0), lambda i: (i, 0, 0, 0)),
            pl.BlockSpec((bo, hp, ho, k0), lambda i: (i, 0, 0, 0)),
            pl.BlockSpec((k1, c1), lambda i: (0, 0)),
            pl.BlockSpec((4, 4 * c1, c2), lambda i: (0, 0, 0)),
        ],
        out_specs=(
            pl.BlockSpec((bo, hh, hh, c2), lambda i: (i, 0, 0, 0)),
            pl.BlockSpec((1, 1, c2), lambda i: (i, 0, 0)),
            pl.BlockSpec((1, 1, c2), lambda i: (i, 0, 0)),
        ),
        out_shape=(
            jax.ShapeDtypeStruct((n, hh, hh, c2), _DT),
            jax.ShapeDtypeStruct((nb, 1, c2), jnp.float32),
            jax.ShapeDtypeStruct((nb, 1, c2), jnp.float32),
        ),
        compiler_params=pltpu.CompilerParams(
            dimension_semantics=("parallel",),
            vmem_limit_bytes=100 * 1024 * 1024,
        ),
        cost_estimate=pl.CostEstimate(flops=flops, transcendentals=0,
                                      bytes_accessed=bytes_acc),
    )(x1a, x1b, w1m, w2t)


def _bnconv_kernel(y_ref, sc_ref, sh_ref, w_ref, o_ref, s_ref, q_ref, *, bo):
    z = y_ref[...]
    z = jnp.maximum(z * sc_ref[0] + sh_ref[0], 0.0).astype(_DT)
    hh = z.shape[1] // 2
    x = _fold_value(z)
    acc = _s2_taps(x, w_ref[...], bo, hh)
    co = acc.shape[-1]
    o_ref[...] = acc.reshape(bo, hh, hh, co).astype(o_ref.dtype)
    s_ref[0, 0] = jnp.sum(acc, axis=0)
    q_ref[0, 0] = jnp.sum(acc * acc, axis=0)


def _bnconv(y, scale, shift, wt, bo):
    n, h, _, c = y.shape
    hh = h // 2
    co = wt.shape[2]
    bo = min(bo, n)
    nb = n // bo
    kern = functools.partial(_bnconv_kernel, bo=bo)
    flops = 2 * n * hh * hh * 4 * c * co
    bytes_acc = 2 * (y.size + wt.size + n * hh * hh * co)
    return pl.pallas_call(
        kern,
        grid=(nb,),
        in_specs=[
            pl.BlockSpec((bo, h, h, c), lambda i: (i, 0, 0, 0)),
            pl.BlockSpec((1, c), lambda i: (0, 0)),
            pl.BlockSpec((1, c), lambda i: (0, 0)),
            pl.BlockSpec((4, 4 * c, co), lambda i: (0, 0, 0)),
        ],
        out_specs=(
            pl.BlockSpec((bo, hh, hh, co), lambda i: (i, 0, 0, 0)),
            pl.BlockSpec((1, 1, co), lambda i: (i, 0, 0)),
            pl.BlockSpec((1, 1, co), lambda i: (i, 0, 0)),
        ),
        out_shape=(
            jax.ShapeDtypeStruct((n, hh, hh, co), _DT),
            jax.ShapeDtypeStruct((nb, 1, co), jnp.float32),
            jax.ShapeDtypeStruct((nb, 1, co), jnp.float32),
        ),
        compiler_params=pltpu.CompilerParams(
            dimension_semantics=("parallel",),
            vmem_limit_bytes=100 * 1024 * 1024,
        ),
        cost_estimate=pl.CostEstimate(flops=flops, transcendentals=0,
                                      bytes_accessed=bytes_acc),
    )(y, scale, shift, wt)


def _conv5_kernel(y_ref, sc_ref, sh_ref, w_ref, o_ref, *, bo):
    z = y_ref[...]
    z = jnp.maximum(z * sc_ref[0] + sh_ref[0], 0.0).astype(_DT)
    h = z.shape[1]
    c = z.shape[-1]
    ho = h - 1
    zp = jnp.pad(z, ((0, 0), (1, 1), (1, 1), (0, 0)))
    m = bo * ho * ho
    co = w_ref.shape[2]
    acc = jnp.zeros((m, co), jnp.float32)
    for t in range(16):
        di, dj = divmod(t, 4)
        tap = zp[:, di:di + ho, dj:dj + ho, :].reshape(m, c)
        acc = acc + jnp.dot(tap, w_ref[t], preferred_element_type=jnp.float32)
    o_ref[...] = acc.reshape(bo, ho, ho, co)


def _conv5(y, scale, shift, wt, bo):
    n, h, _, c = y.shape
    ho = h - 1
    co = wt.shape[2]
    bo = min(bo, n)
    nb = n // bo
    kern = functools.partial(_conv5_kernel, bo=bo)
    flops = 2 * n * ho * ho * 16 * c * co
    bytes_acc = 2 * (y.size + wt.size) + 4 * n * ho * ho * co
    return pl.pallas_call(
        kern,
        grid=(nb,),
        in_specs=[
            pl.BlockSpec((bo, h, h, c), lambda i: (i, 0, 0, 0)),
            pl.BlockSpec((1, c), lambda i: (0, 0)),
            pl.BlockSpec((1, c), lambda i: (0, 0)),
            pl.BlockSpec((16, c, co), lambda i: (0, 0, 0)),
        ],
        out_specs=pl.BlockSpec((bo, ho, ho, co), lambda i: (i, 0, 0, 0)),
        out_shape=jax.ShapeDtypeStruct((n, ho, ho, co), jnp.float32),
        compiler_params=pltpu.CompilerParams(
            dimension_semantics=("parallel",),
            vmem_limit_bytes=100 * 1024 * 1024,
        ),
        cost_estimate=pl.CostEstimate(flops=flops, transcendentals=0,
                                      bytes_accessed=bytes_acc),
    )(y, scale, shift, wt)


def _bn_affine(s, q, count, gamma, beta):
    mean = jnp.sum(s, axis=(0, 1)) / count
    var = jnp.maximum(jnp.sum(q, axis=(0, 1)) / count - mean * mean, 0.0)
    scale = gamma * jax.lax.rsqrt(var + EPS)
    shift = beta - mean * scale
    return scale.reshape(1, -1), shift.reshape(1, -1)


def kernel(x, w1, w2, w3, w4, w5, g2, b2, g3, b3, g4, b4):
    n = x.shape[0]

    x1a, x1b = _host_x1(x)
    y2, s2, q2 = _conv12(x1a, x1b, _pack_w1(w1), _pack_w_s2(w2), 8)
    a2 = _bn_affine(s2, q2, n * y2.shape[1] * y2.shape[2], g2, b2)

    y3, s3, q3 = _bnconv(y2, *a2, _pack_w_s2(w3), 32)
    a3 = _bn_affine(s3, q3, n * y3.shape[1] * y3.shape[2], g3, b3)

    y4, s4, q4 = _bnconv(y3, *a3, _pack_w_s2(w4), 64)
    a4 = _bn_affine(s4, q4, n * y4.shape[1] * y4.shape[2], g4, b4)

    y5 = _conv5(y4, *a4, _pack_w_s1(w5), 64)

    out = y5[..., :1]
    return jnp.transpose(out, (0, 3, 1, 2))


# speedup vs baseline: 3.7235x; 1.0094x over previous
import functools

import jax
import jax.numpy as jnp
from jax.experimental import pallas as pl
from jax.experimental.pallas import tpu as pltpu

EPS = 1e-5
_DT = jnp.bfloat16


def _round_up(v, m):
    return (v + m - 1) // m * m


def _pack_w_s2(w):
    cout, cin, _, _ = w.shape
    wt = jnp.transpose(w, (2, 3, 1, 0))
    wt = wt.reshape(2, 2, 2, 2, cin, cout)
    wt = wt.transpose(0, 2, 1, 3, 4, 5)
    return wt.reshape(4, 4 * cin, cout).astype(_DT)


def _pack_w1(w):
    return _pack_w_s2(w).reshape(-1, w.shape[0])


def _pack_w_s1(w):
    cout, cin, _, _ = w.shape
    coutp = _round_up(cout, 128)
    wt = jnp.transpose(w, (2, 3, 1, 0))
    wt = jnp.pad(wt, ((0, 0), (0, 0), (0, 0), (0, coutp - cout)))
    return wt.reshape(16, cin, coutp).astype(_DT)


def _fold_value(z):
    bo, h, w, c = z.shape
    hh = h // 2
    zp = jnp.pad(z, ((0, 0), (1, 1), (1, 1), (0, 0)))
    v = zp.reshape(bo, hh + 1, 2, hh + 1, 2, c)
    return jnp.concatenate(
        [v[:, :, rp, :, cp, :] for rp in range(2) for cp in range(2)],
        axis=-1)


def _s2_taps(x, wt, bo, hh):
    c4 = x.shape[-1]
    xa = x[:, :, 0:hh, :]
    xb = x[:, :, 1:hh + 1, :]
    m = bo * hh * hh
    acc = jnp.zeros((m, wt.shape[2]), jnp.float32)
    for a in range(2):
        ta = xa[:, a:a + hh].reshape(m, c4)
        tb = xb[:, a:a + hh].reshape(m, c4)
        acc = acc + jnp.dot(ta, wt[a * 2], preferred_element_type=jnp.float32)
        acc = acc + jnp.dot(tb, wt[a * 2 + 1], preferred_element_type=jnp.float32)
    return acc


def _host_x1(x):
    xt = jnp.transpose(x, (0, 2, 3, 1)).astype(_DT)
    n, h, _, c = xt.shape
    hp = h // 2 + 1
    xp = jnp.pad(xt, ((0, 0), (1, 1), (1, 1), (0, 0)))
    x1 = xp.reshape(n, hp, 2, hp, 2, c).transpose(0, 1, 3, 2, 4, 5)
    x1 = x1.reshape(n, hp, hp, 4 * c)
    return x1[:, :, 0:-1, :], x1[:, :, 1:, :]


def _conv12_kernel(x1a_ref, x1b_ref, w1_ref, w2_ref, y_ref, s_ref, q_ref,
                   *, bo, hh, c1, c2):
    ho = 2 * hh
    m1 = bo * ho * ho
    k0 = x1a_ref.shape[-1]
    xa = x1a_ref[...]
    xb = x1b_ref[...]
    patch = jnp.concatenate(
        [(xa if b == 0 else xb)[:, a:a + ho, :, :].reshape(m1, k0)
         for a in range(2) for b in range(2)], axis=-1)
    y1 = jnp.dot(patch, w1_ref[...], preferred_element_type=jnp.float32)
    y1 = jnp.maximum(y1, 0.0).astype(_DT)
    y1 = y1.reshape(bo, ho, ho, c1)

    x2 = _fold_value(y1)
    acc = _s2_taps(x2, w2_ref[...], bo, hh)
    y_ref[...] = acc.reshape(bo, hh, hh, c2).astype(y_ref.dtype)
    s_ref[0, 0] = jnp.sum(acc, axis=0)
    q_ref[0, 0] = jnp.sum(acc * acc, axis=0)


def _conv12(x1a, x1b, w1m, w2t, bo):
    n, hp, ho, k0 = x1a.shape
    hh = ho // 2
    k1, c1 = w1m.shape
    c2 = w2t.shape[2]
    bo = min(bo, n)
    nb = n // bo
    kern = functools.partial(_conv12_kernel, bo=bo, hh=hh, c1=c1, c2=c2)
    flops = 2 * n * ho * ho * k1 * c1 + 2 * n * hh * hh * 4 * c1 * c2
    bytes_acc = 2 * (x1a.size + x1b.size + w1m.size + w2t.size
                     + n * hh * hh * c2)
    return pl.pallas_call(
        kern,
        grid=(nb,),
        in_specs=[
            pl.BlockSpec((bo, hp, ho, k0), lambda i: (i, 0, 0, 0)),
            pl.BlockSpec((bo, hp, ho, k0), lambda i: (i, 0, 0, 0)),
            pl.BlockSpec((k1, c1), lambda i: (0, 0)),
            pl.BlockSpec((4, 4 * c1, c2), lambda i: (0, 0, 0)),
        ],
        out_specs=(
            pl.BlockSpec((bo, hh, hh, c2), lambda i: (i, 0, 0, 0)),
            pl.BlockSpec((1, 1, c2), lambda i: (i, 0, 0)),
            pl.BlockSpec((1, 1, c2), lambda i: (i, 0, 0)),
        ),
        out_shape=(
            jax.ShapeDtypeStruct((n, hh, hh, c2), _DT),
            jax.ShapeDtypeStruct((nb, 1, c2), jnp.float32),
            jax.ShapeDtypeStruct((nb, 1, c2), jnp.float32),
        ),
        compiler_params=pltpu.CompilerParams(
            dimension_semantics=("parallel",),
            vmem_limit_bytes=100 * 1024 * 1024,
        ),
        cost_estimate=pl.CostEstimate(flops=flops, transcendentals=0,
                                      bytes_accessed=bytes_acc),
    )(x1a, x1b, w1m, w2t)


def _bnconv_kernel(y_ref, sc_ref, sh_ref, w_ref, o_ref, s_ref, q_ref, *, bo):
    z = y_ref[...]
    z = jnp.maximum(z * sc_ref[0] + sh_ref[0], 0.0).astype(_DT)
    hh = z.shape[1] // 2
    x = _fold_value(z)
    acc = _s2_taps(x, w_ref[...], bo, hh)
    co = acc.shape[-1]
    o_ref[...] = acc.reshape(bo, hh, hh, co).astype(o_ref.dtype)
    s_ref[0, 0] = jnp.sum(acc, axis=0)
    q_ref[0, 0] = jnp.sum(acc * acc, axis=0)


def _bnconv(y, scale, shift, wt, bo):
    n, h, _, c = y.shape
    hh = h // 2
    co = wt.shape[2]
    bo = min(bo, n)
    nb = n // bo
    kern = functools.partial(_bnconv_kernel, bo=bo)
    flops = 2 * n * hh * hh * 4 * c * co
    bytes_acc = 2 * (y.size + wt.size + n * hh * hh * co)
    return pl.pallas_call(
        kern,
        grid=(nb,),
        in_specs=[
            pl.BlockSpec((bo, h, h, c), lambda i: (i, 0, 0, 0)),
            pl.BlockSpec((1, c), lambda i: (0, 0)),
            pl.BlockSpec((1, c), lambda i: (0, 0)),
            pl.BlockSpec((4, 4 * c, co), lambda i: (0, 0, 0)),
        ],
        out_specs=(
            pl.BlockSpec((bo, hh, hh, co), lambda i: (i, 0, 0, 0)),
            pl.BlockSpec((1, 1, co), lambda i: (i, 0, 0)),
            pl.BlockSpec((1, 1, co), lambda i: (i, 0, 0)),
        ),
        out_shape=(
            jax.ShapeDtypeStruct((n, hh, hh, co), _DT),
            jax.ShapeDtypeStruct((nb, 1, co), jnp.float32),
            jax.ShapeDtypeStruct((nb, 1, co), jnp.float32),
        ),
        compiler_params=pltpu.CompilerParams(
            dimension_semantics=("parallel",),
            vmem_limit_bytes=100 * 1024 * 1024,
        ),
        cost_estimate=pl.CostEstimate(flops=flops, transcendentals=0,
                                      bytes_accessed=bytes_acc),
    )(y, scale, shift, wt)


def _conv5_kernel(y_ref, sc_ref, sh_ref, w_ref, o_ref, *, bo):
    z = y_ref[...]
    z = jnp.maximum(z * sc_ref[0] + sh_ref[0], 0.0).astype(_DT)
    h = z.shape[1]
    c = z.shape[-1]
    ho = h - 1
    zp = jnp.pad(z, ((0, 0), (1, 1), (1, 1), (0, 0)))
    m = bo * ho * ho
    co = w_ref.shape[2]
    acc = jnp.zeros((m, co), jnp.float32)
    for t in range(16):
        di, dj = divmod(t, 4)
        tap = zp[:, di:di + ho, dj:dj + ho, :].reshape(m, c)
        acc = acc + jnp.dot(tap, w_ref[t], preferred_element_type=jnp.float32)
    o_ref[...] = acc.reshape(bo, ho, ho, co)


def _conv5(y, scale, shift, wt, bo):
    n, h, _, c = y.shape
    ho = h - 1
    co = wt.shape[2]
    bo = min(bo, n)
    nb = n // bo
    kern = functools.partial(_conv5_kernel, bo=bo)
    flops = 2 * n * ho * ho * 16 * c * co
    bytes_acc = 2 * (y.size + wt.size) + 4 * n * ho * ho * co
    return pl.pallas_call(
        kern,
        grid=(nb,),
        in_specs=[
            pl.BlockSpec((bo, h, h, c), lambda i: (i, 0, 0, 0)),
            pl.BlockSpec((1, c), lambda i: (0, 0)),
            pl.BlockSpec((1, c), lambda i: (0, 0)),
            pl.BlockSpec((16, c, co), lambda i: (0, 0, 0)),
        ],
        out_specs=pl.BlockSpec((bo, ho, ho, co), lambda i: (i, 0, 0, 0)),
        out_shape=jax.ShapeDtypeStruct((n, ho, ho, co), jnp.float32),
        compiler_params=pltpu.CompilerParams(
            dimension_semantics=("parallel",),
            vmem_limit_bytes=100 * 1024 * 1024,
        ),
        cost_estimate=pl.CostEstimate(flops=flops, transcendentals=0,
                                      bytes_accessed=bytes_acc),
    )(y, scale, shift, wt)


def _bn_affine(s, q, count, gamma, beta):
    mean = jnp.sum(s, axis=(0, 1)) / count
    var = jnp.maximum(jnp.sum(q, axis=(0, 1)) / count - mean * mean, 0.0)
    scale = gamma * jax.lax.rsqrt(var + EPS)
    shift = beta - mean * scale
    return scale.reshape(1, -1), shift.reshape(1, -1)


def kernel(x, w1, w2, w3, w4, w5, g2, b2, g3, b3, g4, b4):
    n = x.shape[0]

    x1a, x1b = _host_x1(x)
    y2, s2, q2 = _conv12(x1a, x1b, _pack_w1(w1), _pack_w_s2(w2), 16)
    a2 = _bn_affine(s2, q2, n * y2.shape[1] * y2.shape[2], g2, b2)

    y3, s3, q3 = _bnconv(y2, *a2, _pack_w_s2(w3), 32)
    a3 = _bn_affine(s3, q3, n * y3.shape[1] * y3.shape[2], g3, b3)

    y4, s4, q4 = _bnconv(y3, *a3, _pack_w_s2(w4), 64)
    a4 = _bn_affine(s4, q4, n * y4.shape[1] * y4.shape[2], g4, b4)

    y5 = _conv5(y4, *a4, _pack_w_s1(w5), 64)

    out = y5[..., :1]
    return jnp.transpose(out, (0, 3, 1, 2))
